# baseline (speedup 1.0000x reference)
import functools

import jax
import jax.numpy as jnp
from jax import lax
from jax.experimental import pallas as pl
from jax.experimental.pallas import tpu as pltpu
from jax.experimental.pallas import tpu_sc as plsc

NUM_EXPERTS = 8
TOP_K = 2
HIDDEN = 1024
INTERMEDIATE = 1024
SWIGLU_LIMIT = 7.0
SWIGLU_ALPHA = 1.702
TOKENS = 1024

NPAIRS = TOKENS * TOP_K
BTR = 128
NB = NPAIRS // BTR + NUM_EXPERTS
NROWS = NB * BTR
NW = 32
PPW = NPAIRS // NW
TPW = TOKENS // NW


def _worker_id():
    return lax.axis_index("s") * 2 + lax.axis_index("c")


def _dispatch_body(hid_hbm, tok_hbm, pos_hbm, xs_hbm,
                   tok_v, pos_v, rows_v, sem):
    wid = _worker_id()
    base = wid * PPW
    pltpu.sync_copy(tok_hbm.at[pl.ds(base, PPW)], tok_v)
    pltpu.sync_copy(pos_hbm.at[pl.ds(base, PPW)], pos_v)
    pltpu.async_copy(hid_hbm.at[tok_v], rows_v, sem).wait()
    pltpu.async_copy(rows_v, xs_hbm.at[pos_v], sem).wait()


def _dispatch_sc(hidden_states, tok, pos):
    mesh = plsc.VectorSubcoreMesh(core_axis_name="c", subcore_axis_name="s")
    return pl.kernel(
        _dispatch_body,
        out_type=jax.ShapeDtypeStruct((NROWS, HIDDEN), jnp.float32),
        mesh=mesh,
        scratch_types=[
            pltpu.VMEM((PPW,), jnp.int32),
            pltpu.VMEM((PPW,), jnp.int32),
            pltpu.VMEM((PPW, HIDDEN), jnp.float32),
            pltpu.SemaphoreType.DMA,
        ],
    )(hidden_states, tok, pos)


def _combine_body(ys_hbm, pos_hbm, out_hbm, pos_v, rows_v, out_v, sem):
    wid = _worker_id()
    pltpu.sync_copy(pos_hbm.at[pl.ds(wid * PPW, PPW)], pos_v)
    pltpu.async_copy(ys_hbm.at[pos_v], rows_v, sem).wait()

    def tok_body(t, carry):
        def chunk_body(j, c):
            sl = pl.ds(j * 16, 16)
            out_v[t, sl] = rows_v[2 * t, sl] + rows_v[2 * t + 1, sl]
            return c
        return lax.fori_loop(0, HIDDEN // 16, chunk_body, carry)

    lax.fori_loop(0, TPW, tok_body, 0)
    pltpu.sync_copy(out_v, out_hbm.at[pl.ds(wid * TPW, TPW)])


def _combine_sc(ys, pos):
    mesh = plsc.VectorSubcoreMesh(core_axis_name="c", subcore_axis_name="s")
    return pl.kernel(
        _combine_body,
        out_type=jax.ShapeDtypeStruct((TOKENS, HIDDEN), jnp.float32),
        mesh=mesh,
        scratch_types=[
            pltpu.VMEM((PPW,), jnp.int32),
            pltpu.VMEM((PPW, HIDDEN), jnp.float32),
            pltpu.VMEM((TPW, HIDDEN), jnp.float32),
            pltpu.SemaphoreType.DMA,
        ],
    )(ys, pos)


def _gmm_body(be_ref, xs_ref, gup_ref, gub_ref, dp2_ref, dpb_ref, wr_ref,
              ys_ref):
    x = xs_ref[...].astype(jnp.bfloat16)
    h = jnp.dot(x, gup_ref[0], preferred_element_type=jnp.float32)
    h += gub_ref[0, 0][None, :]
    a = jnp.minimum(h, SWIGLU_LIMIT)
    a = a * jax.nn.sigmoid(SWIGLU_ALPHA * a)
    b = jnp.clip(h, -SWIGLU_LIMIT, SWIGLU_LIMIT) + 1.0
    s = a * pltpu.roll(b, 2 * INTERMEDIATE - 1, axis=1)
    lane = lax.broadcasted_iota(jnp.int32, s.shape, 1)
    s = jnp.where(lane % 2 == 0, s, 0.0)
    y = jnp.dot(s.astype(jnp.bfloat16), dp2_ref[0],
                preferred_element_type=jnp.float32)
    y += dpb_ref[0, 0][None, :]
    ys_ref[...] = y * wr_ref[0, 0][:, None]


def _gmm_tc(block_expert, xs, gup, gub, dp2, dpb, wrow):
    grid_spec = pltpu.PrefetchScalarGridSpec(
        num_scalar_prefetch=1,
        grid=(NB,),
        in_specs=[
            pl.BlockSpec((BTR, HIDDEN), lambda i, be: (i, 0)),
            pl.BlockSpec((1, HIDDEN, 2 * INTERMEDIATE),
                         lambda i, be: (be[i], 0, 0)),
            pl.BlockSpec((1, 1, 2 * INTERMEDIATE), lambda i, be: (be[i], 0, 0)),
            pl.BlockSpec((1, 2 * INTERMEDIATE, HIDDEN),
                         lambda i, be: (be[i], 0, 0)),
            pl.BlockSpec((1, 1, HIDDEN), lambda i, be: (be[i], 0, 0)),
            pl.BlockSpec((1, 1, BTR), lambda i, be: (i, 0, 0)),
        ],
        out_specs=pl.BlockSpec((BTR, HIDDEN), lambda i, be: (i, 0)),
    )
    return pl.pallas_call(
        _gmm_body,
        grid_spec=grid_spec,
        out_shape=jax.ShapeDtypeStruct((NROWS, HIDDEN), jnp.float32),
    )(block_expert, xs, gup, gub, dp2, dpb, wrow)


def _routing_metadata(router_logits):
    probs = jax.nn.softmax(router_logits, axis=-1)
    topw, topi = lax.top_k(probs, TOP_K)
    topw = topw / jnp.sum(topw, axis=-1, keepdims=True)
    e_flat = topi.reshape(-1).astype(jnp.int32)
    w_flat = topw.reshape(-1)
    onehot = (e_flat[:, None] == jnp.arange(NUM_EXPERTS)[None, :]).astype(
        jnp.int32)
    excl = jnp.cumsum(onehot, axis=0) - onehot
    rank = jnp.take_along_axis(excl, e_flat[:, None], axis=1)[:, 0]
    counts = jnp.sum(onehot, axis=0)
    padded = ((counts + BTR - 1) // BTR) * BTR
    ends = jnp.cumsum(padded)
    offs = ends - padded
    pos = (offs[e_flat] + rank).astype(jnp.int32)
    block_expert = jnp.clip(
        jnp.searchsorted(ends, jnp.arange(NB) * BTR, side='right'),
        0, NUM_EXPERTS - 1).astype(jnp.int32)
    tok = jnp.arange(NPAIRS, dtype=jnp.int32) // TOP_K
    return tok, pos, w_flat, block_expert


def kernel(hidden_states, router_logits, gate_up_proj, gate_up_proj_bias,
           down_proj, down_proj_bias):
    tok, pos, w_flat, block_expert = _routing_metadata(router_logits)
    gup = gate_up_proj.astype(jnp.bfloat16)
    dp2 = jnp.repeat(down_proj, 2, axis=1).astype(jnp.bfloat16)
    wrow = jnp.zeros((NROWS,), jnp.float32).at[pos].set(w_flat)
    xs = _dispatch_sc(hidden_states, tok, pos)
    ys = _gmm_tc(block_expert, xs, gup, gate_up_proj_bias[:, None, :], dp2,
                 down_proj_bias[:, None, :], wrow.reshape(NB, 1, BTR))
    return _combine_sc(ys, pos)


# speedup vs baseline: 1.4740x; 1.4740x over previous
import functools

import jax
import jax.numpy as jnp
from jax import lax
from jax.experimental import pallas as pl
from jax.experimental.pallas import tpu as pltpu
from jax.experimental.pallas import tpu_sc as plsc

NUM_EXPERTS = 8
TOP_K = 2
HIDDEN = 1024
INTERMEDIATE = 1024
SWIGLU_LIMIT = 7.0
SWIGLU_ALPHA = 1.702
TOKENS = 1024

NPAIRS = TOKENS * TOP_K
BTR = 128
NB = NPAIRS // BTR + NUM_EXPERTS
NROWS = NB * BTR
NW = 32
PPW = NPAIRS // NW
TPW = TOKENS // NW


def _worker_id():
    return lax.axis_index("s") * 2 + lax.axis_index("c")


def _dispatch_body(hid_hbm, tok_hbm, pos_hbm, xs_hbm,
                   tok_v, pos_v, rows_v, sem):
    wid = _worker_id()
    base = wid * PPW
    pltpu.sync_copy(tok_hbm.at[pl.ds(base, PPW)], tok_v)
    pltpu.sync_copy(pos_hbm.at[pl.ds(base, PPW)], pos_v)
    pltpu.async_copy(hid_hbm.at[tok_v], rows_v, sem).wait()
    pltpu.async_copy(rows_v, xs_hbm.at[pos_v], sem).wait()


def _dispatch_sc(hidden_states, tok, pos):
    mesh = plsc.VectorSubcoreMesh(core_axis_name="c", subcore_axis_name="s")
    return pl.kernel(
        _dispatch_body,
        out_type=jax.ShapeDtypeStruct((NROWS, HIDDEN), jnp.float32),
        mesh=mesh,
        scratch_types=[
            pltpu.VMEM((PPW,), jnp.int32),
            pltpu.VMEM((PPW,), jnp.int32),
            pltpu.VMEM((PPW, HIDDEN), jnp.float32),
            pltpu.SemaphoreType.DMA,
        ],
    )(hidden_states, tok, pos)


def _combine_body(ys_hbm, pos_hbm, wb_hbm, out_hbm, pos_v, wb_v, rows_v,
                  out_v, sem):
    wid = _worker_id()
    base = wid * PPW
    pltpu.sync_copy(pos_hbm.at[pl.ds(base, PPW)], pos_v)
    pltpu.sync_copy(wb_hbm.at[pl.ds(base, PPW)], wb_v)
    pltpu.async_copy(ys_hbm.at[pos_v], rows_v, sem).wait()

    def tok_body(t, carry):
        def chunk_body(j, c):
            sl = pl.ds(j * 16, 16)
            out_v[t, sl] = (rows_v[2 * t, sl] * wb_v[2 * t, :]
                            + rows_v[2 * t + 1, sl] * wb_v[2 * t + 1, :])
            return c
        return lax.fori_loop(0, HIDDEN // 16, chunk_body, carry)

    lax.fori_loop(0, TPW, tok_body, 0)
    pltpu.sync_copy(out_v, out_hbm.at[pl.ds(wid * TPW, TPW)])


def _combine_sc(ys, pos, wb):
    mesh = plsc.VectorSubcoreMesh(core_axis_name="c", subcore_axis_name="s")
    return pl.kernel(
        _combine_body,
        out_type=jax.ShapeDtypeStruct((TOKENS, HIDDEN), jnp.float32),
        mesh=mesh,
        scratch_types=[
            pltpu.VMEM((PPW,), jnp.int32),
            pltpu.VMEM((PPW, 16), jnp.float32),
            pltpu.VMEM((PPW, HIDDEN), jnp.float32),
            pltpu.VMEM((TPW, HIDDEN), jnp.float32),
            pltpu.SemaphoreType.DMA,
        ],
    )(ys, pos, wb)


def _gmm_body(be_ref, xs_ref, gup_ref, gub_ref, e_ref, dp_ref, dpb_ref,
              ys_ref):
    x = xs_ref[...].astype(jnp.bfloat16)
    h = jnp.dot(x, gup_ref[0], preferred_element_type=jnp.float32)
    h += gub_ref[0, 0][None, :]
    a = jnp.minimum(h, SWIGLU_LIMIT)
    a = a * jax.nn.sigmoid(SWIGLU_ALPHA * a)
    b = jnp.clip(h, -SWIGLU_LIMIT, SWIGLU_LIMIT) + 1.0
    s = a * pltpu.roll(b, 2 * INTERMEDIATE - 1, axis=1)
    u = jnp.dot(s.astype(jnp.bfloat16), e_ref[...],
                preferred_element_type=jnp.float32)
    y = jnp.dot(u.astype(jnp.bfloat16), dp_ref[0],
                preferred_element_type=jnp.float32)
    y += dpb_ref[0, 0][None, :]
    ys_ref[...] = y


def _gmm_tc(block_expert, xs, gup, gub, eye2, dp, dpb):
    grid_spec = pltpu.PrefetchScalarGridSpec(
        num_scalar_prefetch=1,
        grid=(NB,),
        in_specs=[
            pl.BlockSpec((BTR, HIDDEN), lambda i, be: (i, 0)),
            pl.BlockSpec((1, HIDDEN, 2 * INTERMEDIATE),
                         lambda i, be: (be[i], 0, 0)),
            pl.BlockSpec((1, 1, 2 * INTERMEDIATE), lambda i, be: (be[i], 0, 0)),
            pl.BlockSpec((2 * INTERMEDIATE, INTERMEDIATE),
                         lambda i, be: (0, 0)),
            pl.BlockSpec((1, INTERMEDIATE, HIDDEN),
                         lambda i, be: (be[i], 0, 0)),
            pl.BlockSpec((1, 1, HIDDEN), lambda i, be: (be[i], 0, 0)),
        ],
        out_specs=pl.BlockSpec((BTR, HIDDEN), lambda i, be: (i, 0)),
    )
    return pl.pallas_call(
        _gmm_body,
        grid_spec=grid_spec,
        out_shape=jax.ShapeDtypeStruct((NROWS, HIDDEN), jnp.float32),
    )(block_expert, xs, gup, gub, eye2, dp, dpb)


def _routing_metadata(router_logits):
    probs = jax.nn.softmax(router_logits, axis=-1)
    topw, topi = lax.top_k(probs, TOP_K)
    topw = topw / jnp.sum(topw, axis=-1, keepdims=True)
    e_flat = topi.reshape(-1).astype(jnp.int32)
    w_flat = topw.reshape(-1)
    onehot = (e_flat[:, None] == jnp.arange(NUM_EXPERTS)[None, :]).astype(
        jnp.int32)
    excl = jnp.cumsum(onehot, axis=0) - onehot
    rank = jnp.take_along_axis(excl, e_flat[:, None], axis=1)[:, 0]
    counts = jnp.sum(onehot, axis=0)
    padded = ((counts + BTR - 1) // BTR) * BTR
    ends = jnp.cumsum(padded)
    offs = ends - padded
    pos = (offs[e_flat] + rank).astype(jnp.int32)
    block_expert = jnp.clip(
        jnp.searchsorted(ends, jnp.arange(NB) * BTR, side='right'),
        0, NUM_EXPERTS - 1).astype(jnp.int32)
    tok = jnp.arange(NPAIRS, dtype=jnp.int32) // TOP_K
    return tok, pos, w_flat, block_expert


def kernel(hidden_states, router_logits, gate_up_proj, gate_up_proj_bias,
           down_proj, down_proj_bias):
    tok, pos, w_flat, block_expert = _routing_metadata(router_logits)
    gup = gate_up_proj.astype(jnp.bfloat16)
    dp = down_proj.astype(jnp.bfloat16)
    eye2 = (jnp.arange(2 * INTERMEDIATE)[:, None]
            == 2 * jnp.arange(INTERMEDIATE)[None, :]).astype(jnp.bfloat16)
    wb = jnp.broadcast_to(w_flat[:, None], (NPAIRS, 16))
    xs = _dispatch_sc(hidden_states, tok, pos)
    ys = _gmm_tc(block_expert, xs, gup, gate_up_proj_bias[:, None, :], eye2,
                 dp, down_proj_bias[:, None, :])
    return _combine_sc(ys, pos, wb)


# speedup vs baseline: 1.5192x; 1.0307x over previous
import functools

import jax
import jax.numpy as jnp
from jax import lax
from jax.experimental import pallas as pl
from jax.experimental.pallas import tpu as pltpu
from jax.experimental.pallas import tpu_sc as plsc

NUM_EXPERTS = 8
TOP_K = 2
HIDDEN = 1024
INTERMEDIATE = 1024
SWIGLU_LIMIT = 7.0
SWIGLU_ALPHA = 1.702
TOKENS = 1024

NPAIRS = TOKENS * TOP_K
BTR = 128
NB = NPAIRS // BTR + NUM_EXPERTS
NROWS = NB * BTR
NW = 32
PPW = NPAIRS // NW
TPW = TOKENS // NW


def _worker_id():
    return lax.axis_index("s") * 2 + lax.axis_index("c")


def _dispatch_body(hid_hbm, tok_hbm, pos_hbm, xs_hbm,
                   tok_v, pos_v, rows_v, sem):
    wid = _worker_id()
    base = wid * PPW
    pltpu.sync_copy(tok_hbm.at[pl.ds(base, PPW)], tok_v)
    pltpu.sync_copy(pos_hbm.at[pl.ds(base, PPW)], pos_v)
    pltpu.async_copy(hid_hbm.at[tok_v], rows_v, sem).wait()
    pltpu.async_copy(rows_v, xs_hbm.at[pos_v], sem).wait()


def _dispatch_sc(hidden_states, tok, pos):
    mesh = plsc.VectorSubcoreMesh(core_axis_name="c", subcore_axis_name="s")
    return pl.kernel(
        _dispatch_body,
        out_type=jax.ShapeDtypeStruct((NROWS, HIDDEN), jnp.float32),
        mesh=mesh,
        scratch_types=[
            pltpu.VMEM((PPW,), jnp.int32),
            pltpu.VMEM((PPW,), jnp.int32),
            pltpu.VMEM((PPW, HIDDEN), jnp.float32),
            pltpu.SemaphoreType.DMA,
        ],
    )(hidden_states, tok, pos)


def _combine_body(ys_hbm, pos_hbm, wb_hbm, out_hbm, pos_v, wb_v, rows_v,
                  out_v, sem):
    wid = _worker_id()
    base = wid * PPW
    pltpu.sync_copy(pos_hbm.at[pl.ds(base, PPW)], pos_v)
    pltpu.sync_copy(wb_hbm.at[pl.ds(base, PPW)], wb_v)
    pltpu.async_copy(ys_hbm.at[pos_v], rows_v, sem).wait()

    def tok_body(t, carry):
        def chunk_body(j, c):
            sl = pl.ds(j * 16, 16)
            out_v[t, sl] = (rows_v[2 * t, sl] * wb_v[2 * t, :]
                            + rows_v[2 * t + 1, sl] * wb_v[2 * t + 1, :])
            return c
        return lax.fori_loop(0, HIDDEN // 16, chunk_body, carry)

    lax.fori_loop(0, TPW, tok_body, 0)
    pltpu.sync_copy(out_v, out_hbm.at[pl.ds(wid * TPW, TPW)])


def _combine_sc(ys, pos, wb):
    mesh = plsc.VectorSubcoreMesh(core_axis_name="c", subcore_axis_name="s")
    return pl.kernel(
        _combine_body,
        out_type=jax.ShapeDtypeStruct((TOKENS, HIDDEN), jnp.float32),
        mesh=mesh,
        scratch_types=[
            pltpu.VMEM((PPW,), jnp.int32),
            pltpu.VMEM((PPW, 16), jnp.float32),
            pltpu.VMEM((PPW, HIDDEN), jnp.float32),
            pltpu.VMEM((TPW, HIDDEN), jnp.float32),
            pltpu.SemaphoreType.DMA,
        ],
    )(ys, pos, wb)


def _gmm_body(be_ref, lv_ref, xs_ref, gup_ref, gub_ref, e_ref, dp_ref,
              dpb_ref, ys_ref):
    i = pl.program_id(0)

    @pl.when(lv_ref[i] != 0)
    def _():
        x = xs_ref[...].astype(jnp.bfloat16)
        h = jnp.dot(x, gup_ref[0], preferred_element_type=jnp.float32)
        h += gub_ref[0, 0][None, :]
        a = jnp.minimum(h, SWIGLU_LIMIT)
        a = a * jax.nn.sigmoid(SWIGLU_ALPHA * a)
        b = jnp.clip(h, -SWIGLU_LIMIT, SWIGLU_LIMIT) + 1.0
        s = a * pltpu.roll(b, 2 * INTERMEDIATE - 1, axis=1)
        u = jnp.dot(s.astype(jnp.bfloat16), e_ref[...],
                    preferred_element_type=jnp.float32)
        y = jnp.dot(u.astype(jnp.bfloat16), dp_ref[0],
                    preferred_element_type=jnp.float32)
        y += dpb_ref[0, 0][None, :]
        ys_ref[...] = y


def _gmm_tc(block_expert, live, xs, gup, gub, eye2, dp, dpb):
    grid_spec = pltpu.PrefetchScalarGridSpec(
        num_scalar_prefetch=2,
        grid=(NB,),
        in_specs=[
            pl.BlockSpec((BTR, HIDDEN), lambda i, be, lv: (i, 0)),
            pl.BlockSpec((1, HIDDEN, 2 * INTERMEDIATE),
                         lambda i, be, lv: (be[i], 0, 0)),
            pl.BlockSpec((1, 1, 2 * INTERMEDIATE),
                         lambda i, be, lv: (be[i], 0, 0)),
            pl.BlockSpec((2 * INTERMEDIATE, INTERMEDIATE),
                         lambda i, be, lv: (0, 0)),
            pl.BlockSpec((1, INTERMEDIATE, HIDDEN),
                         lambda i, be, lv: (be[i], 0, 0)),
            pl.BlockSpec((1, 1, HIDDEN), lambda i, be, lv: (be[i], 0, 0)),
        ],
        out_specs=pl.BlockSpec((BTR, HIDDEN), lambda i, be, lv: (i, 0)),
    )
    return pl.pallas_call(
        _gmm_body,
        grid_spec=grid_spec,
        out_shape=jax.ShapeDtypeStruct((NROWS, HIDDEN), jnp.float32),
    )(block_expert, live, xs, gup, gub, eye2, dp, dpb)


def _routing_metadata(router_logits):
    probs = jax.nn.softmax(router_logits, axis=-1)
    topw, topi = lax.top_k(probs, TOP_K)
    topw = topw / jnp.sum(topw, axis=-1, keepdims=True)
    e_flat = topi.reshape(-1).astype(jnp.int32)
    w_flat = topw.reshape(-1)
    onehot = (e_flat[:, None] == jnp.arange(NUM_EXPERTS)[None, :]).astype(
        jnp.int32)
    excl = jnp.cumsum(onehot, axis=0) - onehot
    rank = jnp.take_along_axis(excl, e_flat[:, None], axis=1)[:, 0]
    counts = jnp.sum(onehot, axis=0)
    padded = ((counts + BTR - 1) // BTR) * BTR
    ends = jnp.cumsum(padded)
    offs = ends - padded
    pos = (offs[e_flat] + rank).astype(jnp.int32)
    block_expert = jnp.clip(
        jnp.searchsorted(ends, jnp.arange(NB) * BTR, side='right'),
        0, NUM_EXPERTS - 1).astype(jnp.int32)
    live = (jnp.arange(NB, dtype=jnp.int32) * BTR < ends[-1]).astype(jnp.int32)
    tok = jnp.arange(NPAIRS, dtype=jnp.int32) // TOP_K
    return tok, pos, w_flat, block_expert, live


def kernel(hidden_states, router_logits, gate_up_proj, gate_up_proj_bias,
           down_proj, down_proj_bias):
    tok, pos, w_flat, block_expert, live = _routing_metadata(router_logits)
    gup = gate_up_proj.astype(jnp.bfloat16)
    dp = down_proj.astype(jnp.bfloat16)
    eye2 = (jnp.arange(2 * INTERMEDIATE)[:, None]
            == 2 * jnp.arange(INTERMEDIATE)[None, :]).astype(jnp.bfloat16)
    wb = jnp.broadcast_to(w_flat[:, None], (NPAIRS, 16))
    xs = _dispatch_sc(hidden_states, tok, pos)
    ys = _gmm_tc(block_expert, live, xs, gup, gate_up_proj_bias[:, None, :],
                 eye2, dp, down_proj_bias[:, None, :])
    return _combine_sc(ys, pos, wb)


# speedup vs baseline: 1.9374x; 1.2753x over previous
import functools

import jax
import jax.numpy as jnp
from jax import lax
from jax.experimental import pallas as pl
from jax.experimental.pallas import tpu as pltpu
from jax.experimental.pallas import tpu_sc as plsc

NUM_EXPERTS = 8
TOP_K = 2
HIDDEN = 1024
INTERMEDIATE = 1024
SWIGLU_LIMIT = 7.0
SWIGLU_ALPHA = 1.702
TOKENS = 1024

NPAIRS = TOKENS * TOP_K
BTR = 128
NB = NPAIRS // BTR + NUM_EXPERTS
NROWS = NB * BTR
NW = 32
PPW = NPAIRS // NW
TPW = TOKENS // NW


def _worker_id():
    return lax.axis_index("s") * 2 + lax.axis_index("c")


def _dispatch_body(hid_hbm, tok_hbm, pos_hbm, xs_hbm,
                   tok_v, pos_v, rows_v, sem):
    wid = _worker_id()
    base = wid * PPW
    pltpu.sync_copy(tok_hbm.at[pl.ds(base, PPW)], tok_v)
    pltpu.sync_copy(pos_hbm.at[pl.ds(base, PPW)], pos_v)
    pltpu.async_copy(hid_hbm.at[tok_v], rows_v, sem).wait()
    pltpu.async_copy(rows_v, xs_hbm.at[pos_v], sem).wait()


def _dispatch_sc(hidden_states, tok, pos):
    mesh = plsc.VectorSubcoreMesh(core_axis_name="c", subcore_axis_name="s")
    return pl.kernel(
        _dispatch_body,
        out_type=jax.ShapeDtypeStruct((NROWS, HIDDEN), jnp.float32),
        mesh=mesh,
        scratch_types=[
            pltpu.VMEM((PPW,), jnp.int32),
            pltpu.VMEM((PPW,), jnp.int32),
            pltpu.VMEM((PPW, HIDDEN), jnp.float32),
            pltpu.SemaphoreType.DMA,
        ],
    )(hidden_states, tok, pos)


def _combine_body(ys_hbm, pos_hbm, wb_hbm, out_hbm, pos_v, wb_v, rows_v,
                  out_v, sem):
    wid = _worker_id()
    base = wid * PPW
    pltpu.sync_copy(pos_hbm.at[pl.ds(base, PPW)], pos_v)
    pltpu.sync_copy(wb_hbm.at[pl.ds(base, PPW)], wb_v)
    pltpu.async_copy(ys_hbm.at[pos_v], rows_v, sem).wait()

    def tok_body(t, carry):
        def chunk_body(j, c):
            sl = pl.ds(j * 16, 16)
            out_v[t, sl] = (rows_v[2 * t, sl] * wb_v[2 * t, :]
                            + rows_v[2 * t + 1, sl] * wb_v[2 * t + 1, :])
            return c
        return lax.fori_loop(0, HIDDEN // 16, chunk_body, carry)

    lax.fori_loop(0, TPW, tok_body, 0)
    pltpu.sync_copy(out_v, out_hbm.at[pl.ds(wid * TPW, TPW)])


def _combine_sc(ys, pos, wb):
    mesh = plsc.VectorSubcoreMesh(core_axis_name="c", subcore_axis_name="s")
    return pl.kernel(
        _combine_body,
        out_type=jax.ShapeDtypeStruct((TOKENS, HIDDEN), jnp.float32),
        mesh=mesh,
        scratch_types=[
            pltpu.VMEM((PPW,), jnp.int32),
            pltpu.VMEM((PPW, 16), jnp.float32),
            pltpu.VMEM((PPW, HIDDEN), jnp.float32),
            pltpu.VMEM((TPW, HIDDEN), jnp.float32),
            pltpu.SemaphoreType.DMA,
        ],
    )(ys, pos, wb)


def _gmm_body(be_ref, lv_ref, xs_ref, gup_ref, gub_ref, e_ref, dp_ref,
              dpb_ref, ys_ref):
    i = pl.program_id(0)

    @pl.when(lv_ref[i] != 0)
    def _():
        x = xs_ref[...].astype(jnp.bfloat16)
        h = jnp.dot(x, gup_ref[0].astype(jnp.bfloat16),
                    preferred_element_type=jnp.float32)
        h += gub_ref[0, 0][None, :]
        a = jnp.minimum(h, SWIGLU_LIMIT)
        a = a * jax.nn.sigmoid(SWIGLU_ALPHA * a)
        b = jnp.clip(h, -SWIGLU_LIMIT, SWIGLU_LIMIT) + 1.0
        s = a * pltpu.roll(b, 2 * INTERMEDIATE - 1, axis=1)
        u = jnp.dot(s.astype(jnp.bfloat16), e_ref[...],
                    preferred_element_type=jnp.float32)
        y = jnp.dot(u.astype(jnp.bfloat16), dp_ref[0].astype(jnp.bfloat16),
                    preferred_element_type=jnp.float32)
        y += dpb_ref[0, 0][None, :]
        ys_ref[...] = y


def _gmm_tc(block_expert, live, xs, gup, gub, eye2, dp, dpb):
    grid_spec = pltpu.PrefetchScalarGridSpec(
        num_scalar_prefetch=2,
        grid=(NB,),
        in_specs=[
            pl.BlockSpec((BTR, HIDDEN), lambda i, be, lv: (i, 0)),
            pl.BlockSpec((1, HIDDEN, 2 * INTERMEDIATE),
                         lambda i, be, lv: (be[i], 0, 0)),
            pl.BlockSpec((1, 1, 2 * INTERMEDIATE),
                         lambda i, be, lv: (be[i], 0, 0)),
            pl.BlockSpec((2 * INTERMEDIATE, INTERMEDIATE),
                         lambda i, be, lv: (0, 0)),
            pl.BlockSpec((1, INTERMEDIATE, HIDDEN),
                         lambda i, be, lv: (be[i], 0, 0)),
            pl.BlockSpec((1, 1, HIDDEN), lambda i, be, lv: (be[i], 0, 0)),
        ],
        out_specs=pl.BlockSpec((BTR, HIDDEN), lambda i, be, lv: (i, 0)),
    )
    return pl.pallas_call(
        _gmm_body,
        grid_spec=grid_spec,
        out_shape=jax.ShapeDtypeStruct((NROWS, HIDDEN), jnp.float32),
    )(block_expert, live, xs, gup, gub, eye2, dp, dpb)


def _routing_metadata(router_logits):
    probs = jax.nn.softmax(router_logits, axis=-1)
    topw, topi = lax.top_k(probs, TOP_K)
    topw = topw / jnp.sum(topw, axis=-1, keepdims=True)
    e_flat = topi.reshape(-1).astype(jnp.int32)
    w_flat = topw.reshape(-1)
    onehot = (e_flat[:, None] == jnp.arange(NUM_EXPERTS)[None, :]).astype(
        jnp.int32)
    excl = jnp.cumsum(onehot, axis=0) - onehot
    rank = jnp.take_along_axis(excl, e_flat[:, None], axis=1)[:, 0]
    counts = jnp.sum(onehot, axis=0)
    padded = ((counts + BTR - 1) // BTR) * BTR
    ends = jnp.cumsum(padded)
    offs = ends - padded
    pos = (offs[e_flat] + rank).astype(jnp.int32)
    block_expert = jnp.clip(
        jnp.searchsorted(ends, jnp.arange(NB) * BTR, side='right'),
        0, NUM_EXPERTS - 1).astype(jnp.int32)
    live = (jnp.arange(NB, dtype=jnp.int32) * BTR < ends[-1]).astype(jnp.int32)
    tok = jnp.arange(NPAIRS, dtype=jnp.int32) // TOP_K
    return tok, pos, w_flat, block_expert, live


def kernel(hidden_states, router_logits, gate_up_proj, gate_up_proj_bias,
           down_proj, down_proj_bias):
    tok, pos, w_flat, block_expert, live = _routing_metadata(router_logits)
    eye2 = (jnp.arange(2 * INTERMEDIATE)[:, None]
            == 2 * jnp.arange(INTERMEDIATE)[None, :]).astype(jnp.bfloat16)
    wb = jnp.broadcast_to(w_flat[:, None], (NPAIRS, 16))
    xs = _dispatch_sc(hidden_states, tok, pos)
    ys = _gmm_tc(block_expert, live, xs, gate_up_proj,
                 gate_up_proj_bias[:, None, :], eye2, down_proj,
                 down_proj_bias[:, None, :])
    return _combine_sc(ys, pos, wb)
